# baseline (speedup 1.0000x reference)
import jax
import jax.numpy as jnp
from jax import lax
from jax.experimental import pallas as pl
from jax.experimental.pallas import tpu as pltpu
from jax.experimental.pallas import tpu_sc as plsc

EMB_DIM = 64
BATCH = 16384
N_SERVICE = 1000

NUM_CORES = 2
NUM_SUBCORES = 16
NUM_WORKERS = NUM_CORES * NUM_SUBCORES

ROWS_PER_WORKER = BATCH // NUM_WORKERS
CHUNK = 128
CHUNKS = ROWS_PER_WORKER // CHUNK
TBLOCK = 64


def _stage_table(t_hbm, tbuf, shared, col0):
    pltpu.sync_copy(t_hbm.at[:, pl.ds(col0, TBLOCK)],
                    tbuf.at[pl.ds(0, EMB_DIM), :])

    @pl.loop(0, TBLOCK)
    def row(e):
        for g in range(EMB_DIM // 16):
            vals = plsc.load_gather(
                tbuf, [lax.iota(jnp.int32, 16) + 16 * g,
                       jnp.full((16,), 0, jnp.int32) + e])
            tbuf[EMB_DIM + e, pl.ds(16 * g, 16)] = vals
    pltpu.sync_copy(tbuf.at[pl.ds(EMB_DIM, TBLOCK), :],
                    shared.at[pl.ds(col0, TBLOCK)])


def _gather_body(serv_hbm, genr_hbm, idx_hbm, out_hbm,
                 idx_v, sbuf, gbuf, tbuf, shared_s, shared_g, gsems, wsem):
    wid = lax.axis_index("s") * NUM_CORES + lax.axis_index("c")
    sid = lax.axis_index("s")
    col0 = jnp.minimum(sid * TBLOCK, N_SERVICE - TBLOCK)
    _stage_table(serv_hbm, tbuf, shared_s, col0)
    _stage_table(genr_hbm, tbuf, shared_g, col0)
    plsc.subcore_barrier()

    ib = pl.multiple_of(wid * 2 * CHUNKS, 2 * CHUNKS)
    ob = pl.multiple_of(wid * ROWS_PER_WORKER, ROWS_PER_WORKER)
    pltpu.sync_copy(idx_hbm.at[pl.ds(ib, 2 * CHUNKS)], idx_v)
    gathers = []
    for j in range(CHUNKS):
        rows = pl.ds(j * CHUNK, CHUNK)
        gathers.append(pltpu.async_copy(
            shared_s.at[idx_v.at[2 * j]], sbuf.at[rows, :],
            gsems.at[2 * j]))
        gathers.append(pltpu.async_copy(
            shared_g.at[idx_v.at[2 * j + 1]], gbuf.at[rows, :],
            gsems.at[2 * j + 1]))
    for g in gathers:
        g.wait()
    pltpu.sync_copy(sbuf, out_hbm.at[pl.ds(ob, ROWS_PER_WORKER),
                                     pl.ds(0, EMB_DIM)])
    pltpu.sync_copy(gbuf, out_hbm.at[pl.ds(ob, ROWS_PER_WORKER),
                                     pl.ds(EMB_DIM, EMB_DIM)])


@jax.jit
def _gather(serv_t, genr_t, idx):
    mesh = plsc.VectorSubcoreMesh(core_axis_name="c", subcore_axis_name="s")
    k = pl.kernel(
        _gather_body,
        out_type=jax.ShapeDtypeStruct((BATCH, 2 * EMB_DIM), jnp.float32),
        mesh=mesh,
        scratch_types=[
            pltpu.VMEM((2 * CHUNKS, CHUNK), jnp.int32),
            pltpu.VMEM((ROWS_PER_WORKER, EMB_DIM), jnp.float32),
            pltpu.VMEM((ROWS_PER_WORKER, EMB_DIM), jnp.float32),
            pltpu.VMEM((EMB_DIM + TBLOCK, TBLOCK), jnp.float32),
            pltpu.VMEM_SHARED((N_SERVICE, EMB_DIM), jnp.float32),
            pltpu.VMEM_SHARED((N_SERVICE, EMB_DIM), jnp.float32),
            pltpu.SemaphoreType.DMA((2 * CHUNKS,)),
            pltpu.SemaphoreType.DMA,
        ],
        compiler_params=pltpu.CompilerParams(use_tc_tiling_on_sc=False,
                                             needs_layout_passes=False),
    )
    return k(serv_t, genr_t, idx)


def kernel(x2, emb_service, emb_genre):
    xi = x2.astype(jnp.int32)
    idx = xi.T.reshape(2, BATCH // CHUNK, CHUNK).transpose(1, 0, 2)
    idx = idx.reshape(2 * BATCH // CHUNK, CHUNK)
    return _gather(emb_service.T, emb_genre.T, idx)


# speedup vs baseline: 1.2142x; 1.2142x over previous
import jax
import jax.numpy as jnp
from jax import lax
from jax.experimental import pallas as pl
from jax.experimental.pallas import tpu as pltpu
from jax.experimental.pallas import tpu_sc as plsc

EMB_DIM = 64
BATCH = 16384
N_SERVICE = 1000

NUM_CORES = 2
NUM_SUBCORES = 16
NUM_WORKERS = NUM_CORES * NUM_SUBCORES

ROWS_PER_WORKER = BATCH // NUM_WORKERS
CHUNK = 128
CHUNKS = ROWS_PER_WORKER // CHUNK


def _gather_body(serv_hbm, genr_hbm, idx_hbm, out_hbm,
                 idx_v, sbuf, gbuf, shared_s, shared_g, gsems, wsem):
    wid = lax.axis_index("s") * NUM_CORES + lax.axis_index("c")
    sid = lax.axis_index("s")
    srow = sid * (N_SERVICE // NUM_SUBCORES)
    nrows = N_SERVICE // NUM_SUBCORES + 8
    srow = jnp.minimum(srow, N_SERVICE - nrows)
    pltpu.sync_copy(serv_hbm.at[pl.ds(srow, nrows)],
                    shared_s.at[pl.ds(srow, nrows)])
    pltpu.sync_copy(genr_hbm.at[pl.ds(srow, nrows)],
                    shared_g.at[pl.ds(srow, nrows)])
    plsc.subcore_barrier()
    ib = pl.multiple_of(wid * 2 * CHUNKS, 2 * CHUNKS)
    ob = pl.multiple_of(wid * ROWS_PER_WORKER, ROWS_PER_WORKER)
    pltpu.sync_copy(idx_hbm.at[pl.ds(ib, 2 * CHUNKS)], idx_v)
    gathers = []
    for j in range(CHUNKS):
        rows = pl.ds(j * CHUNK, CHUNK)
        gathers.append(pltpu.async_copy(
            shared_s.at[idx_v.at[2 * j]], sbuf.at[rows, :],
            gsems.at[2 * j]))
        gathers.append(pltpu.async_copy(
            shared_g.at[idx_v.at[2 * j + 1]], gbuf.at[rows, :],
            gsems.at[2 * j + 1]))
    for g in gathers:
        g.wait()
    pltpu.sync_copy(sbuf, out_hbm.at[pl.ds(ob, ROWS_PER_WORKER),
                                     pl.ds(0, EMB_DIM)])
    pltpu.sync_copy(gbuf, out_hbm.at[pl.ds(ob, ROWS_PER_WORKER),
                                     pl.ds(EMB_DIM, EMB_DIM)])


@jax.jit
def _gather(serv, genr, idx):
    mesh = plsc.VectorSubcoreMesh(core_axis_name="c", subcore_axis_name="s")
    k = pl.kernel(
        _gather_body,
        out_type=jax.ShapeDtypeStruct((BATCH, 2 * EMB_DIM), jnp.float32),
        mesh=mesh,
        scratch_types=[
            pltpu.VMEM((2 * CHUNKS, CHUNK), jnp.int32),
            pltpu.VMEM((ROWS_PER_WORKER, EMB_DIM), jnp.float32),
            pltpu.VMEM((ROWS_PER_WORKER, EMB_DIM), jnp.float32),
            pltpu.VMEM_SHARED((N_SERVICE, EMB_DIM), jnp.float32),
            pltpu.VMEM_SHARED((N_SERVICE, EMB_DIM), jnp.float32),
            pltpu.SemaphoreType.DMA((2 * CHUNKS,)),
            pltpu.SemaphoreType.DMA,
        ],
        compiler_params=pltpu.CompilerParams(use_tc_tiling_on_sc=False),
    )
    return k(serv, genr, idx)


def kernel(x2, emb_service, emb_genre):
    xi = x2.astype(jnp.int32)
    idx = xi.T.reshape(2, BATCH // CHUNK, CHUNK).transpose(1, 0, 2)
    idx = idx.reshape(2 * BATCH // CHUNK, CHUNK)
    return _gather(emb_service, emb_genre, idx)


# speedup vs baseline: 1.2669x; 1.0434x over previous
import jax
import jax.numpy as jnp
from jax import lax
from jax.experimental import pallas as pl
from jax.experimental.pallas import tpu as pltpu
from jax.experimental.pallas import tpu_sc as plsc

EMB_DIM = 64
BATCH = 16384
N_SERVICE = 1000

NUM_CORES = 2
NUM_SUBCORES = 16
NUM_WORKERS = NUM_CORES * NUM_SUBCORES

ROWS_PER_WORKER = BATCH // NUM_WORKERS
CHUNK = 128
CHUNKS = ROWS_PER_WORKER // CHUNK


def _gather_body(tbl_hbm, idx_hbm, out_hbm,
                 idx_v, sbuf, gbuf, shared_tbl, gsems, wsem):
    wid = lax.axis_index("s") * NUM_CORES + lax.axis_index("c")
    sid = lax.axis_index("s")
    srow = sid * (2 * N_SERVICE // NUM_SUBCORES)
    pltpu.sync_copy(tbl_hbm.at[pl.ds(srow, 2 * N_SERVICE // NUM_SUBCORES)],
                    shared_tbl.at[pl.ds(srow, 2 * N_SERVICE // NUM_SUBCORES)])
    plsc.subcore_barrier()
    ib = pl.multiple_of(wid * 2 * CHUNKS, 2 * CHUNKS)
    ob = pl.multiple_of(wid * ROWS_PER_WORKER, ROWS_PER_WORKER)
    pltpu.sync_copy(idx_hbm.at[pl.ds(ib, 2 * CHUNKS)], idx_v)
    off = jnp.full((16,), N_SERVICE, jnp.int32)
    for j in range(CHUNKS):
        for c in range(CHUNK // 16):
            sl = (2 * j + 1, pl.ds(c * 16, 16))
            idx_v[sl] = idx_v[sl] + off
    gathers = []
    for j in range(CHUNKS):
        rows = pl.ds(j * CHUNK, CHUNK)
        gathers.append(pltpu.async_copy(
            shared_tbl.at[idx_v.at[2 * j]], sbuf.at[rows, :],
            gsems.at[2 * j]))
        gathers.append(pltpu.async_copy(
            shared_tbl.at[idx_v.at[2 * j + 1]], gbuf.at[rows, :],
            gsems.at[2 * j + 1]))
    for g in gathers:
        g.wait()
    pltpu.sync_copy(sbuf, out_hbm.at[pl.ds(ob, ROWS_PER_WORKER),
                                     pl.ds(0, EMB_DIM)])
    pltpu.sync_copy(gbuf, out_hbm.at[pl.ds(ob, ROWS_PER_WORKER),
                                     pl.ds(EMB_DIM, EMB_DIM)])


@jax.jit
def _gather(tbl, idx):
    mesh = plsc.VectorSubcoreMesh(core_axis_name="c", subcore_axis_name="s")
    k = pl.kernel(
        _gather_body,
        out_type=jax.ShapeDtypeStruct((BATCH, 2 * EMB_DIM), jnp.float32),
        mesh=mesh,
        scratch_types=[
            pltpu.VMEM((2 * CHUNKS, CHUNK), jnp.int32),
            pltpu.VMEM((ROWS_PER_WORKER, EMB_DIM), jnp.float32),
            pltpu.VMEM((ROWS_PER_WORKER, EMB_DIM), jnp.float32),
            pltpu.VMEM_SHARED((2 * N_SERVICE, EMB_DIM), jnp.float32),
            pltpu.SemaphoreType.DMA((2 * CHUNKS,)),
            pltpu.SemaphoreType.DMA,
        ],
        compiler_params=pltpu.CompilerParams(use_tc_tiling_on_sc=False),
    )
    return k(tbl, idx)


def kernel(x2, emb_service, emb_genre):
    xi = x2.astype(jnp.int32)
    idx = xi.T.reshape(2, BATCH // CHUNK, CHUNK).transpose(1, 0, 2)
    idx = idx.reshape(2 * BATCH // CHUNK, CHUNK)
    tbl = jnp.concatenate((emb_service, emb_genre), axis=0)
    return _gather(tbl, idx)


# speedup vs baseline: 1.3556x; 1.0700x over previous
import jax
import jax.numpy as jnp
from jax import lax
from jax.experimental import pallas as pl
from jax.experimental.pallas import tpu as pltpu
from jax.experimental.pallas import tpu_sc as plsc

EMB_DIM = 64
BATCH = 16384
N_SERVICE = 1000

NUM_CORES = 2
NUM_SUBCORES = 16
NUM_WORKERS = NUM_CORES * NUM_SUBCORES

ROWS_PER_WORKER = BATCH // NUM_WORKERS
CHUNK = 128
CHUNKS = ROWS_PER_WORKER // CHUNK


def _gather_body(tbl_hbm, idx_hbm, out_hbm,
                 idx_v, sbuf, gbuf, shared_tbl, gsems, wsem):
    wid = lax.axis_index("s") * NUM_CORES + lax.axis_index("c")
    sid = lax.axis_index("s")
    srow = sid * (2 * N_SERVICE // NUM_SUBCORES)
    stage = pltpu.async_copy(
        tbl_hbm.at[pl.ds(srow, 2 * N_SERVICE // NUM_SUBCORES)],
        shared_tbl.at[pl.ds(srow, 2 * N_SERVICE // NUM_SUBCORES)], wsem)
    ib = pl.multiple_of(wid * 2 * CHUNKS, 2 * CHUNKS)
    ob = pl.multiple_of(wid * ROWS_PER_WORKER, ROWS_PER_WORKER)
    pltpu.sync_copy(idx_hbm.at[pl.ds(ib, 2 * CHUNKS)], idx_v)
    stage.wait()
    plsc.subcore_barrier()
    off = jnp.full((16,), N_SERVICE, jnp.int32)
    for j in range(CHUNKS):
        for c in range(CHUNK // 16):
            sl = (2 * j + 1, pl.ds(c * 16, 16))
            idx_v[sl] = idx_v[sl] + off
    gathers = []
    for j in range(CHUNKS):
        rows = pl.ds(j * CHUNK, CHUNK)
        gathers.append(pltpu.async_copy(
            shared_tbl.at[idx_v.at[2 * j]], sbuf.at[rows, :],
            gsems.at[2 * j]))
        gathers.append(pltpu.async_copy(
            shared_tbl.at[idx_v.at[2 * j + 1]], gbuf.at[rows, :],
            gsems.at[2 * j + 1]))
    writes = []
    for j in range(CHUNKS):
        rows = pl.ds(j * CHUNK, CHUNK)
        orows = pl.ds(ob + j * CHUNK, CHUNK)
        gathers[2 * j].wait()
        writes.append(pltpu.async_copy(
            sbuf.at[rows, :], out_hbm.at[orows, pl.ds(0, EMB_DIM)], wsem))
        gathers[2 * j + 1].wait()
        writes.append(pltpu.async_copy(
            gbuf.at[rows, :], out_hbm.at[orows, pl.ds(EMB_DIM, EMB_DIM)],
            wsem))
    for w in writes:
        w.wait()


@jax.jit
def _gather(tbl, idx):
    mesh = plsc.VectorSubcoreMesh(core_axis_name="c", subcore_axis_name="s")
    k = pl.kernel(
        _gather_body,
        out_type=jax.ShapeDtypeStruct((BATCH, 2 * EMB_DIM), jnp.float32),
        mesh=mesh,
        scratch_types=[
            pltpu.VMEM((2 * CHUNKS, CHUNK), jnp.int32),
            pltpu.VMEM((ROWS_PER_WORKER, EMB_DIM), jnp.float32),
            pltpu.VMEM((ROWS_PER_WORKER, EMB_DIM), jnp.float32),
            pltpu.VMEM_SHARED((2 * N_SERVICE, EMB_DIM), jnp.float32),
            pltpu.SemaphoreType.DMA((2 * CHUNKS,)),
            pltpu.SemaphoreType.DMA,
        ],
        compiler_params=pltpu.CompilerParams(use_tc_tiling_on_sc=False),
    )
    return k(tbl, idx)


def kernel(x2, emb_service, emb_genre):
    xi = x2.astype(jnp.int32)
    idx = xi.T.reshape(2, BATCH // CHUNK, CHUNK).transpose(1, 0, 2)
    idx = idx.reshape(2 * BATCH // CHUNK, CHUNK)
    tbl = jnp.concatenate((emb_service, emb_genre), axis=0)
    return _gather(tbl, idx)
